# baseline (speedup 1.0000x reference)
import functools

import jax
import jax.numpy as jnp
from jax import lax
from jax.experimental import pallas as pl
from jax.experimental.pallas import tpu as pltpu
from jax.experimental.pallas import tpu_sc as plsc

_N = 50000
_E = 800000
_BLK = 512
_EBLK = 800
_GCHUNK = 128
_NCHUNKS = _E // _GCHUNK
_NW = 32


def _fg_kernel(h_ref, wd_ref, ws_ref, ba_ref, f_ref, g_ref):
    h = h_ref[...]
    f_ref[...] = jnp.dot(h, wd_ref[...], preferred_element_type=jnp.float32, precision=lax.Precision.HIGHEST) + ba_ref[...]
    g_ref[...] = jnp.dot(h, ws_ref[...], preferred_element_type=jnp.float32, precision=lax.Precision.HIGHEST)


def _node_fg(h, Wd, Ws, ba):
    n, c = h.shape
    d = Wd.shape[1]
    grid = (n + _BLK - 1) // _BLK
    full = lambda arr: pl.BlockSpec(arr.shape, lambda i: (0,) * arr.ndim)
    return pl.pallas_call(
        _fg_kernel,
        grid=(grid,),
        in_specs=[pl.BlockSpec((_BLK, c), lambda i: (i, 0)),
                  full(Wd), full(Ws), full(ba)],
        out_specs=[pl.BlockSpec((_BLK, d), lambda i: (i, 0)),
                   pl.BlockSpec((_BLK, d), lambda i: (i, 0))],
        out_shape=[jax.ShapeDtypeStruct((n, d), jnp.float32),
                   jax.ShapeDtypeStruct((n, d), jnp.float32)],
    )(h, Wd, Ws, ba)


def _gather_body(f_hbm, g_hbm, dst_hbm, src_hbm, fd_hbm, gs_hbm,
                 idxd_v, idxs_v, rowsf_v, rowsg_v, semf, semg):
    wid = lax.axis_index("s") * 2 + lax.axis_index("c")
    nchunks = jnp.where(wid < _NCHUNKS % _NW, _NCHUNKS // _NW + 1, _NCHUNKS // _NW)

    def body(i, _):
        base = (wid + i * _NW) * _GCHUNK
        pltpu.sync_copy(dst_hbm.at[pl.ds(base, _GCHUNK)], idxd_v)
        pltpu.sync_copy(src_hbm.at[pl.ds(base, _GCHUNK)], idxs_v)
        cpf = pltpu.async_copy(f_hbm.at[idxd_v], rowsf_v, semf)
        cpg = pltpu.async_copy(g_hbm.at[idxs_v], rowsg_v, semg)
        cpf.wait()
        pltpu.sync_copy(rowsf_v, fd_hbm.at[pl.ds(base, _GCHUNK)])
        cpg.wait()
        pltpu.sync_copy(rowsg_v, gs_hbm.at[pl.ds(base, _GCHUNK)])
        return _

    lax.fori_loop(0, nchunks, body, 0)


def _sc_gather(F, G, dsts, srcs):
    d = F.shape[1]
    mesh = plsc.VectorSubcoreMesh(core_axis_name="c", subcore_axis_name="s")
    out_t = [jax.ShapeDtypeStruct((_E, d), jnp.float32),
             jax.ShapeDtypeStruct((_E, d), jnp.float32)]
    scratch = [
        pltpu.VMEM((_GCHUNK,), jnp.int32),
        pltpu.VMEM((_GCHUNK,), jnp.int32),
        pltpu.VMEM((_GCHUNK, d), jnp.float32),
        pltpu.VMEM((_GCHUNK, d), jnp.float32),
        pltpu.SemaphoreType.DMA,
        pltpu.SemaphoreType.DMA,
    ]
    return pl.kernel(_gather_body, out_type=out_t, mesh=mesh,
                     scratch_types=scratch)(F, G, dsts, srcs)


def _edge_mm_kernel(fd_ref, gs_ref, wb_ref, bb_ref, u_ref):
    t = jnp.maximum(fd_ref[...] + gs_ref[...], 0.0)
    u_ref[...] = jnp.dot(t, wb_ref[...], preferred_element_type=jnp.float32, precision=lax.Precision.HIGHEST) + bb_ref[...]


def _edge_mm(FD, GS, Wb, bb):
    d = Wb.shape[1]
    grid = _E // _EBLK
    full = lambda arr: pl.BlockSpec(arr.shape, lambda i: (0,) * arr.ndim)
    return pl.pallas_call(
        _edge_mm_kernel,
        grid=(grid,),
        in_specs=[pl.BlockSpec((_EBLK, FD.shape[1]), lambda i: (i, 0)),
                  pl.BlockSpec((_EBLK, GS.shape[1]), lambda i: (i, 0)),
                  full(Wb), full(bb)],
        out_specs=pl.BlockSpec((_EBLK, d), lambda i: (i, 0)),
        out_shape=jax.ShapeDtypeStruct((_E, d), jnp.float32),
    )(FD, GS, Wb, bb)


def _mlp_head_kernel(h1_ref, h2_ref, h3_ref,
                     w1a_ref, w1b_ref, w1c_ref, b1_ref,
                     w2_ref, b2_ref, w3_ref, b3_ref, out_ref):
    a = (jnp.dot(h1_ref[...], w1a_ref[...], preferred_element_type=jnp.float32, precision=lax.Precision.HIGHEST)
         + jnp.dot(h2_ref[...], w1b_ref[...], preferred_element_type=jnp.float32, precision=lax.Precision.HIGHEST)
         + jnp.dot(h3_ref[...], w1c_ref[...], preferred_element_type=jnp.float32, precision=lax.Precision.HIGHEST)
         + b1_ref[...])
    a = jnp.maximum(a, 0.0)
    b = jnp.maximum(jnp.dot(a, w2_ref[...], preferred_element_type=jnp.float32, precision=lax.Precision.HIGHEST)
                    + b2_ref[...], 0.0)
    out_ref[...] = jnp.dot(b, w3_ref[...], preferred_element_type=jnp.float32, precision=lax.Precision.HIGHEST) + b3_ref[...]


def _mlp_head(h1, h2, h3, Wm1, bm1, Wm2, bm2, Wm3, bm3):
    n = h1.shape[0]
    grid = (n + _BLK - 1) // _BLK
    w1a, w1b, w1c = Wm1[:64], Wm1[64:192], Wm1[192:]
    full = lambda arr: pl.BlockSpec(arr.shape, lambda i: (0,) * arr.ndim)
    return pl.pallas_call(
        _mlp_head_kernel,
        grid=(grid,),
        in_specs=[
            pl.BlockSpec((_BLK, 64), lambda i: (i, 0)),
            pl.BlockSpec((_BLK, 128), lambda i: (i, 0)),
            pl.BlockSpec((_BLK, 256), lambda i: (i, 0)),
            full(w1a), full(w1b), full(w1c), full(bm1),
            full(Wm2), full(bm2), full(Wm3), full(bm3),
        ],
        out_specs=pl.BlockSpec((_BLK, 4), lambda i: (i, 0)),
        out_shape=jax.ShapeDtypeStruct((n, 4), jnp.float32),
    )(h1, h2, h3, w1a, w1b, w1c, bm1, Wm2, bm2, Wm3, bm3)


def _edge_layer(h, srcs, dsts, Wa, ba, Wb, bb):
    c = h.shape[1]
    Wd, Ws, Wb_p = Wa[:c] - Wa[c:], Wa[c:], Wb
    if Wa.shape[1] < 128:
        pad = 128 - Wa.shape[1]
        Wd = jnp.pad(Wd, ((0, 0), (0, pad)))
        Ws = jnp.pad(Ws, ((0, 0), (0, pad)))
        ba = jnp.pad(ba, (0, pad))
        Wb_p = jnp.pad(Wb, ((0, pad), (0, 0)))
    F, G = _node_fg(h, Wd, Ws, ba)
    FD, GS = _sc_gather(F, G, dsts, srcs)
    u = _edge_mm(FD, GS, Wb_p, bb)
    agg = jax.ops.segment_max(u, dsts, num_segments=_N, indices_are_sorted=True)
    return jnp.where(jnp.isneginf(agg), 0.0, agg)


def kernel(x, edge_index, batch,
           W1a, b1a, W1b, b1b,
           W2a, b2a, W2b, b2b,
           W3a, b3a, W3b, b3b,
           Wm1, bm1, Wm2, bm2, Wm3, bm3):
    dsts, srcs = jax.lax.sort((edge_index[1], edge_index[0]), num_keys=1)
    h1 = _edge_layer(x, srcs, dsts, W1a, b1a, W1b, b1b)
    h2 = _edge_layer(h1, srcs, dsts, W2a, b2a, W2b, b2b)
    h3 = _edge_layer(h2, srcs, dsts, W3a, b3a, W3b, b3b)
    return _mlp_head(h1, h2, h3, Wm1, bm1, Wm2, bm2, Wm3, bm3)


# speedup vs baseline: 1.0738x; 1.0738x over previous
import functools

import jax
import jax.numpy as jnp
from jax import lax
from jax.experimental import pallas as pl
from jax.experimental.pallas import tpu as pltpu
from jax.experimental.pallas import tpu_sc as plsc

_N = 50000
_E = 800000
_NW = 32
_BLK = 512
_EBLK = 1024
_GCHUNK = 64
_WEDGES = 25088
_EPAD = _WEDGES * _NW
_HALF = _WEDGES // 4
_DSENT = 53248


def _fg_kernel(h_ref, wd_ref, ws_ref, ba_ref, f_ref, g_ref):
    h = h_ref[...]
    f_ref[...] = jnp.dot(h, wd_ref[...], preferred_element_type=jnp.float32, precision=lax.Precision.HIGHEST) + ba_ref[...]
    g_ref[...] = jnp.dot(h, ws_ref[...], preferred_element_type=jnp.float32, precision=lax.Precision.HIGHEST)


def _node_fg(h, Wd, Ws, ba):
    n, c = h.shape
    d = Wd.shape[1]
    grid = (n + _BLK - 1) // _BLK
    full = lambda arr: pl.BlockSpec(arr.shape, lambda i: (0,) * arr.ndim)
    return pl.pallas_call(
        _fg_kernel,
        grid=(grid,),
        in_specs=[pl.BlockSpec((_BLK, c), lambda i: (i, 0)),
                  full(Wd), full(Ws), full(ba)],
        out_specs=[pl.BlockSpec((_BLK, d), lambda i: (i, 0)),
                   pl.BlockSpec((_BLK, d), lambda i: (i, 0))],
        out_shape=[jax.ShapeDtypeStruct((n, d), jnp.float32),
                   jax.ShapeDtypeStruct((n, d), jnp.float32)],
    )(h, Wd, Ws, ba)


def _gather_body(f_hbm, g_hbm, dst_hbm, src_hbm, fd_hbm, gs_hbm,
                 idxd_v, idxs_v, rf0, rg0, rf1, rg1,
                 semf0, semg0, semf1, semg1):
    wid = lax.axis_index("s") * 2 + lax.axis_index("c")
    wbase = wid * _WEDGES

    def issue(co, rf, rg, sf, sg):
        cf = pltpu.async_copy(f_hbm.at[idxd_v.at[pl.ds(co * _GCHUNK, _GCHUNK)]],
                              rf, sf)
        cg = pltpu.async_copy(g_hbm.at[idxs_v.at[pl.ds(co * _GCHUNK, _GCHUNK)]],
                              rg, sg)
        return cf, cg

    def drain_store(co, hbase, rf, rg, sf, sg):
        pltpu.make_async_copy(f_hbm.at[idxd_v.at[pl.ds(co * _GCHUNK, _GCHUNK)]],
                              rf, sf).wait()
        pltpu.sync_copy(rf, fd_hbm.at[pl.ds(hbase + co * _GCHUNK, _GCHUNK)])
        pltpu.make_async_copy(g_hbm.at[idxs_v.at[pl.ds(co * _GCHUNK, _GCHUNK)]],
                              rg, sg).wait()
        pltpu.sync_copy(rg, gs_hbm.at[pl.ds(hbase + co * _GCHUNK, _GCHUNK)])

    nc = _HALF // _GCHUNK
    for h in range(4):
        hbase = wbase + h * _HALF
        pltpu.sync_copy(dst_hbm.at[pl.ds(hbase, _HALF)], idxd_v)
        pltpu.sync_copy(src_hbm.at[pl.ds(hbase, _HALF)], idxs_v)
        issue(0, rf0, rg0, semf0, semg0)

        def pair(j, _):
            c0 = 2 * j
            issue(c0 + 1, rf1, rg1, semf1, semg1)
            drain_store(c0, hbase, rf0, rg0, semf0, semg0)

            @pl.when(j < nc // 2 - 1)
            def _issue_next():
                issue(c0 + 2, rf0, rg0, semf0, semg0)

            drain_store(c0 + 1, hbase, rf1, rg1, semf1, semg1)
            return _

        lax.fori_loop(0, nc // 2, pair, 0)


def _sc_gather(F, G, dsts, srcs):
    d = F.shape[1]
    mesh = plsc.VectorSubcoreMesh(core_axis_name="c", subcore_axis_name="s")
    out_t = [jax.ShapeDtypeStruct((_EPAD, d), jnp.float32),
             jax.ShapeDtypeStruct((_EPAD, d), jnp.float32)]
    scratch = [
        pltpu.VMEM((_HALF,), jnp.int32),
        pltpu.VMEM((_HALF,), jnp.int32),
        pltpu.VMEM((_GCHUNK, d), jnp.float32),
        pltpu.VMEM((_GCHUNK, d), jnp.float32),
        pltpu.VMEM((_GCHUNK, d), jnp.float32),
        pltpu.VMEM((_GCHUNK, d), jnp.float32),
        pltpu.SemaphoreType.DMA,
        pltpu.SemaphoreType.DMA,
        pltpu.SemaphoreType.DMA,
        pltpu.SemaphoreType.DMA,
    ]
    return pl.kernel(_gather_body, out_type=out_t, mesh=mesh,
                     scratch_types=scratch)(F, G, dsts, srcs)


def _edge_mm_kernel(fd_ref, gs_ref, wb_ref, bb_ref, u_ref):
    t = jnp.maximum(fd_ref[...] + gs_ref[...], 0.0)
    u_ref[...] = jnp.dot(t, wb_ref[...], preferred_element_type=jnp.float32, precision=lax.Precision.HIGHEST) + bb_ref[...]


def _edge_mm(FD, GS, Wb, bb):
    d = Wb.shape[1]
    grid = _EPAD // _EBLK
    full = lambda arr: pl.BlockSpec(arr.shape, lambda i: (0,) * arr.ndim)
    return pl.pallas_call(
        _edge_mm_kernel,
        grid=(grid,),
        in_specs=[pl.BlockSpec((_EBLK, FD.shape[1]), lambda i: (i, 0)),
                  pl.BlockSpec((_EBLK, GS.shape[1]), lambda i: (i, 0)),
                  full(Wb), full(bb)],
        out_specs=pl.BlockSpec((_EBLK, d), lambda i: (i, 0)),
        out_shape=jax.ShapeDtypeStruct((_EPAD, d), jnp.float32),
    )(FD, GS, Wb, bb)


def _mlp_head_kernel(h1_ref, h2_ref, h3_ref,
                     w1a_ref, w1b_ref, w1c_ref, b1_ref,
                     w2_ref, b2_ref, w3_ref, b3_ref, out_ref):
    a = (jnp.dot(h1_ref[...], w1a_ref[...], preferred_element_type=jnp.float32, precision=lax.Precision.HIGHEST)
         + jnp.dot(h2_ref[...], w1b_ref[...], preferred_element_type=jnp.float32, precision=lax.Precision.HIGHEST)
         + jnp.dot(h3_ref[...], w1c_ref[...], preferred_element_type=jnp.float32, precision=lax.Precision.HIGHEST)
         + b1_ref[...])
    a = jnp.maximum(a, 0.0)
    b = jnp.maximum(jnp.dot(a, w2_ref[...], preferred_element_type=jnp.float32, precision=lax.Precision.HIGHEST)
                    + b2_ref[...], 0.0)
    out_ref[...] = jnp.dot(b, w3_ref[...], preferred_element_type=jnp.float32, precision=lax.Precision.HIGHEST) + b3_ref[...]


def _mlp_head(h1, h2, h3, Wm1, bm1, Wm2, bm2, Wm3, bm3):
    n = h1.shape[0]
    grid = (n + _BLK - 1) // _BLK
    w1a, w1b, w1c = Wm1[:64], Wm1[64:192], Wm1[192:]
    full = lambda arr: pl.BlockSpec(arr.shape, lambda i: (0,) * arr.ndim)
    return pl.pallas_call(
        _mlp_head_kernel,
        grid=(grid,),
        in_specs=[
            pl.BlockSpec((_BLK, 64), lambda i: (i, 0)),
            pl.BlockSpec((_BLK, 128), lambda i: (i, 0)),
            pl.BlockSpec((_BLK, 256), lambda i: (i, 0)),
            full(w1a), full(w1b), full(w1c), full(bm1),
            full(Wm2), full(bm2), full(Wm3), full(bm3),
        ],
        out_specs=pl.BlockSpec((_BLK, 4), lambda i: (i, 0)),
        out_shape=jax.ShapeDtypeStruct((n, 4), jnp.float32),
    )(h1, h2, h3, w1a, w1b, w1c, bm1, Wm2, bm2, Wm3, bm3)


def _edge_layer(h, srcs, dsts, Wa, ba, Wb, bb):
    c = h.shape[1]
    Wd, Ws, Wb_p = Wa[:c] - Wa[c:], Wa[c:], Wb
    if Wa.shape[1] < 128:
        pad = 128 - Wa.shape[1]
        Wd = jnp.pad(Wd, ((0, 0), (0, pad)))
        Ws = jnp.pad(Ws, ((0, 0), (0, pad)))
        ba = jnp.pad(ba, (0, pad))
        Wb_p = jnp.pad(Wb, ((0, pad), (0, 0)))
    F, G = _node_fg(h, Wd, Ws, ba)
    FD, GS = _sc_gather(F, G, jnp.where(dsts >= _N, 0, dsts), srcs)
    u = _edge_mm(FD, GS, Wb_p, bb)
    agg = jax.ops.segment_max(u, dsts, num_segments=_N, indices_are_sorted=True)
    return jnp.where(jnp.isneginf(agg), 0.0, agg)


def kernel(x, edge_index, batch,
           W1a, b1a, W1b, b1b,
           W2a, b2a, W2b, b2b,
           W3a, b3a, W3b, b3b,
           Wm1, bm1, Wm2, bm2, Wm3, bm3):
    dsts, srcs = jax.lax.sort((edge_index[1], edge_index[0]), num_keys=1)
    dsts = jnp.concatenate([dsts, jnp.full((_EPAD - _E,), _DSENT, jnp.int32)])
    srcs = jnp.concatenate([srcs, jnp.zeros((_EPAD - _E,), jnp.int32)])
    h1 = _edge_layer(x, srcs, dsts, W1a, b1a, W1b, b1b)
    h2 = _edge_layer(h1, srcs, dsts, W2a, b2a, W2b, b2b)
    h3 = _edge_layer(h2, srcs, dsts, W3a, b3a, W3b, b3b)
    return _mlp_head(h1, h2, h3, Wm1, bm1, Wm2, bm2, Wm3, bm3)
